# Initial kernel scaffold; baseline (speedup 1.0000x reference)
import jax
import jax.numpy as jnp
from jax.experimental import pallas as pl


def kernel(action_indices, table):
    raise NotImplementedError("write your pallas kernel here")


# speedup vs baseline: 2.3966x; 2.3966x over previous
import functools

import jax
import jax.numpy as jnp
from jax import lax
from jax.experimental import pallas as pl
from jax.experimental.pallas import tpu as pltpu
from jax.experimental.pallas import tpu_sc as plsc

NUM_CORES = 2
NUM_SUBCORES = 16
NUM_WORKERS = NUM_CORES * NUM_SUBCORES
CHUNK = 128


@functools.lru_cache(maxsize=None)
def _build_lookup(n_chunks, vocab, dim):
    b_per_w = n_chunks * CHUNK
    total = NUM_WORKERS * b_per_w
    mesh = plsc.VectorSubcoreMesh(
        core_axis_name="c",
        subcore_axis_name="s",
        num_cores=NUM_CORES,
        num_subcores=NUM_SUBCORES,
    )

    @functools.partial(
        pl.kernel,
        out_type=jax.ShapeDtypeStruct((total, dim), jnp.float32),
        mesh=mesh,
        scratch_types=[
            pltpu.VMEM((n_chunks, CHUNK), jnp.int32),
            pltpu.VMEM((2, CHUNK, dim), jnp.float32),
            pltpu.SemaphoreType.DMA,
            pltpu.SemaphoreType.DMA,
        ],
    )
    def lookup(idx_hbm, table_hbm, out_hbm, idx_v, rows_v, gsem, osem):
        wid = lax.axis_index("s") * NUM_CORES + lax.axis_index("c")
        base = wid * b_per_w
        pltpu.sync_copy(idx_hbm.at[wid], idx_v)
        pltpu.async_copy(table_hbm.at[idx_v.at[0]], rows_v.at[0], gsem)

        @pl.loop(0, n_chunks, step=2)
        def _(j0):
            for b in range(2):
                j = j0 + b
                nb = 1 - b
                pltpu.make_async_copy(
                    table_hbm.at[idx_v.at[j]], rows_v.at[b], gsem
                ).wait()
                @pl.when(j >= 1)
                def _():
                    pltpu.make_async_copy(
                        rows_v.at[nb],
                        out_hbm.at[pl.ds(base, CHUNK)],
                        osem,
                    ).wait()

                @pl.when(j + 1 < n_chunks)
                def _():
                    pltpu.async_copy(
                        table_hbm.at[idx_v.at[j + 1]], rows_v.at[nb], gsem
                    )

                pltpu.async_copy(
                    rows_v.at[b],
                    out_hbm.at[pl.ds(base + j * CHUNK, CHUNK)],
                    osem,
                )

        pltpu.make_async_copy(
            rows_v.at[0], out_hbm.at[pl.ds(base, CHUNK)], osem
        ).wait()

    return lookup


def kernel(action_indices, table):
    n, k = action_indices.shape
    vocab, dim = table.shape
    total = n * k
    assert total % (NUM_WORKERS * CHUNK) == 0
    n_chunks = total // (NUM_WORKERS * CHUNK)
    idx = action_indices.astype(jnp.int32).reshape(NUM_WORKERS, n_chunks, CHUNK)
    out = _build_lookup(n_chunks, vocab, dim)(idx, table)
    return out.reshape(n, k, dim)


# speedup vs baseline: 2.4260x; 1.0122x over previous
import functools

import jax
import jax.numpy as jnp
from jax import lax
from jax.experimental import pallas as pl
from jax.experimental.pallas import tpu as pltpu
from jax.experimental.pallas import tpu_sc as plsc

NUM_CORES = 2
NUM_SUBCORES = 16
NUM_WORKERS = NUM_CORES * NUM_SUBCORES
CHUNK = 128


@functools.lru_cache(maxsize=None)
def _build_lookup(n_chunks, vocab, dim):
    b_per_w = n_chunks * CHUNK
    total = NUM_WORKERS * b_per_w
    mesh = plsc.VectorSubcoreMesh(
        core_axis_name="c",
        subcore_axis_name="s",
        num_cores=NUM_CORES,
        num_subcores=NUM_SUBCORES,
    )

    NBUF = 3
    assert n_chunks >= NBUF
    main = (n_chunks - 2) // NBUF * NBUF

    @functools.partial(
        pl.kernel,
        out_type=jax.ShapeDtypeStruct((total, dim), jnp.float32),
        mesh=mesh,
        scratch_types=[
            pltpu.VMEM((n_chunks, CHUNK), jnp.int32),
            pltpu.VMEM((NBUF, CHUNK, dim), jnp.float32),
            pltpu.SemaphoreType.DMA,
            pltpu.SemaphoreType.DMA,
        ],
    )
    def lookup(idx_hbm, table_hbm, out_hbm, idx_v, rows_v, gsem, osem):
        wid = lax.axis_index("s") * NUM_CORES + lax.axis_index("c")
        base = wid * b_per_w
        pltpu.sync_copy(idx_hbm.at[wid], idx_v)

        def gather(j, slot):
            pltpu.async_copy(table_hbm.at[idx_v.at[j]], rows_v.at[slot], gsem)

        def wait_gather(slot):
            pltpu.make_async_copy(
                table_hbm.at[idx_v.at[0]], rows_v.at[slot], gsem
            ).wait()

        def put(j, slot):
            pltpu.async_copy(
                rows_v.at[slot], out_hbm.at[pl.ds(base + j * CHUNK, CHUNK)], osem
            )

        def wait_put(slot):
            pltpu.make_async_copy(
                rows_v.at[slot], out_hbm.at[pl.ds(base, CHUNK)], osem
            ).wait()

        gather(0, 0)
        gather(1, 1)

        @pl.loop(0, main, step=NBUF)
        def _(j0):
            for b in range(NBUF):
                j = j0 + b
                nxt = (b + 2) % NBUF
                wait_gather(b)
                @pl.when(j >= 1)
                def _():
                    wait_put(nxt)

                gather(j + 2, nxt)
                put(j, b)

        @pl.loop(main, n_chunks)
        def _(j):
            for b in range(NBUF):
                @pl.when(j % NBUF == b)
                def _():
                    wait_gather(b)
                    wait_put((b + 2) % NBUF)
                    put(j, b)

        wait_put(0)

    return lookup


def kernel(action_indices, table):
    n, k = action_indices.shape
    vocab, dim = table.shape
    total = n * k
    assert total % (NUM_WORKERS * CHUNK) == 0
    n_chunks = total // (NUM_WORKERS * CHUNK)
    idx = action_indices.astype(jnp.int32).reshape(NUM_WORKERS, n_chunks, CHUNK)
    out = _build_lookup(n_chunks, vocab, dim)(idx, table)
    return out.reshape(n, k, dim)


# speedup vs baseline: 7.4555x; 3.0732x over previous
import functools

import jax
import jax.numpy as jnp
from jax import lax
from jax.experimental import pallas as pl
from jax.experimental.pallas import tpu as pltpu
from jax.experimental.pallas import tpu_sc as plsc

NUM_CORES = 2
NUM_SUBCORES = 16
NUM_WORKERS = NUM_CORES * NUM_SUBCORES
CHUNK = 128


@functools.lru_cache(maxsize=None)
def _build_lookup(n_chunks, vocab, dim):
    b_per_w = n_chunks * CHUNK
    total = NUM_WORKERS * b_per_w
    mesh = plsc.VectorSubcoreMesh(
        core_axis_name="c",
        subcore_axis_name="s",
        num_cores=NUM_CORES,
        num_subcores=NUM_SUBCORES,
    )

    NBUF = 3
    assert n_chunks >= NBUF
    main = (n_chunks - 2) // NBUF * NBUF

    @functools.partial(
        pl.kernel,
        out_type=jax.ShapeDtypeStruct((total, dim), jnp.float32),
        mesh=mesh,
        scratch_types=[
            pltpu.VMEM((n_chunks, CHUNK), jnp.int32),
            pltpu.VMEM((NBUF, CHUNK, dim), jnp.float32),
            pltpu.SemaphoreType.DMA,
            pltpu.SemaphoreType.DMA,
        ],
    )
    def lookup(idx_hbm, table_hbm, out_hbm, idx_v, rows_v, gsem, osem):
        wid = lax.axis_index("s") * NUM_CORES + lax.axis_index("c")
        base = wid * b_per_w
        pltpu.sync_copy(idx_hbm.at[wid], idx_v)

        def gather(j, slot):
            pltpu.async_copy(table_hbm.at[idx_v.at[j]], rows_v.at[slot], gsem)

        def wait_gather(slot):
            pltpu.make_async_copy(
                table_hbm.at[idx_v.at[0]], rows_v.at[slot], gsem
            ).wait()

        def put(j, slot):
            pltpu.async_copy(
                rows_v.at[slot], out_hbm.at[pl.ds(base + j * CHUNK, CHUNK)], osem
            )

        def wait_put(slot):
            pltpu.make_async_copy(
                rows_v.at[slot], out_hbm.at[pl.ds(base, CHUNK)], osem
            ).wait()

        gather(0, 0)
        gather(1, 1)

        @pl.loop(0, main, step=NBUF)
        def _(j0):
            for b in range(NBUF):
                j = j0 + b
                nxt = (b + 2) % NBUF
                wait_gather(b)
                @pl.when(j >= 1)
                def _():
                    wait_put(nxt)

                gather(j + 2, nxt)
                put(j, b)

        @pl.loop(main, n_chunks)
        def _(j):
            for b in range(NBUF):
                @pl.when(j % NBUF == b)
                def _():
                    wait_gather(b)
                    wait_put((b + 2) % NBUF)
                    put(j, b)

        wait_put(0)

    return lookup


def kernel(action_indices, table):
    n, k = action_indices.shape
    vocab, dim = table.shape
    total = n * k
    assert total % (NUM_WORKERS * CHUNK) == 0
    n_chunks = total // (NUM_WORKERS * CHUNK)
    idx = action_indices.astype(jnp.int32).T.reshape(NUM_WORKERS, n_chunks, CHUNK)
    out = _build_lookup(n_chunks, vocab, dim)(idx, table)
    return out.reshape(k, n, dim).transpose(1, 0, 2)


# speedup vs baseline: 7.4817x; 1.0035x over previous
import functools

import jax
import jax.numpy as jnp
from jax import lax
from jax.experimental import pallas as pl
from jax.experimental.pallas import tpu as pltpu
from jax.experimental.pallas import tpu_sc as plsc

NUM_CORES = 2
NUM_SUBCORES = 16
NUM_WORKERS = NUM_CORES * NUM_SUBCORES
CHUNK = 80
NBUF = 4
GDEPTH = 2


@functools.lru_cache(maxsize=None)
def _build_lookup(n_chunks, vocab, dim):
    b_per_w = n_chunks * CHUNK
    total = NUM_WORKERS * b_per_w
    mesh = plsc.VectorSubcoreMesh(
        core_axis_name="c",
        subcore_axis_name="s",
        num_cores=NUM_CORES,
        num_subcores=NUM_SUBCORES,
    )
    lag = NBUF - GDEPTH
    assert n_chunks % NBUF == 0 and n_chunks >= NBUF and 0 < lag < NBUF

    @functools.partial(
        pl.kernel,
        out_type=jax.ShapeDtypeStruct((total, dim), jnp.float32),
        mesh=mesh,
        scratch_types=[
            pltpu.VMEM((n_chunks, CHUNK), jnp.int32),
            pltpu.VMEM((NBUF, CHUNK, dim), jnp.float32),
            pltpu.SemaphoreType.DMA,
            pltpu.SemaphoreType.DMA,
        ],
    )
    def lookup(idx_hbm, table_hbm, out_hbm, idx_v, rows_v, gsem, osem):
        wid = lax.axis_index("s") * NUM_CORES + lax.axis_index("c")
        base = wid * b_per_w
        pltpu.sync_copy(idx_hbm.at[wid], idx_v)

        def gather(j, slot):
            pltpu.async_copy(table_hbm.at[idx_v.at[j]], rows_v.at[slot], gsem)

        def wait_gather(slot):
            pltpu.make_async_copy(
                table_hbm.at[idx_v.at[0]], rows_v.at[slot], gsem
            ).wait()

        def put(j, slot):
            pltpu.async_copy(
                rows_v.at[slot], out_hbm.at[pl.ds(base + j * CHUNK, CHUNK)], osem
            )

        def wait_put(slot):
            pltpu.make_async_copy(
                rows_v.at[slot], out_hbm.at[pl.ds(base, CHUNK)], osem
            ).wait()

        for j in range(GDEPTH):
            gather(j, j)

        @pl.loop(0, n_chunks, step=NBUF)
        def _(j0):
            for b in range(NBUF):
                j = j0 + b
                nxt = (b + GDEPTH) % NBUF
                wait_gather(b)
                @pl.when(j >= lag)
                def _():
                    wait_put(nxt)

                @pl.when(j + GDEPTH < n_chunks)
                def _():
                    gather(j + GDEPTH, nxt)

                put(j, b)

        for _ in range(lag):
            wait_put(0)

    return lookup


def kernel(action_indices, table):
    n, k = action_indices.shape
    vocab, dim = table.shape
    total = n * k
    assert total % (NUM_WORKERS * CHUNK) == 0
    n_chunks = total // (NUM_WORKERS * CHUNK)
    idx = action_indices.astype(jnp.int32).T.reshape(NUM_WORKERS, n_chunks, CHUNK)
    out = _build_lookup(n_chunks, vocab, dim)(idx, table)
    return out.reshape(k, n, dim).transpose(1, 0, 2)
